# baseline (speedup 1.0000x reference)
import jax
import jax.numpy as jnp
from jax import lax
from jax.experimental import pallas as pl
from jax.experimental.pallas import tpu as pltpu

NBUF = 8
ROWS = 1176


def _qsample_body(t_sm, c1t, c2t, x_hbm, n_hbm, o_hbm,
                  xb, nb, ob, xsem, nsem, osem):
    i = pl.program_id(0)
    num = pl.num_programs(0)
    rows_per_batch = x_hbm.shape[1]
    cpb = rows_per_batch // ROWS

    def in_copies(step):
        slot = lax.rem(step, NBUF)
        b = step // cpb
        r0 = lax.rem(step, cpb) * ROWS
        cx = pltpu.make_async_copy(
            x_hbm.at[b, pl.ds(r0, ROWS), :], xb.at[slot], xsem.at[slot])
        cn = pltpu.make_async_copy(
            n_hbm.at[b, pl.ds(r0, ROWS), :], nb.at[slot], nsem.at[slot])
        return cx, cn

    def out_copy(step):
        slot = lax.rem(step, NBUF)
        b = step // cpb
        r0 = lax.rem(step, cpb) * ROWS
        return pltpu.make_async_copy(
            ob.at[slot], o_hbm.at[b, pl.ds(r0, ROWS), :], osem.at[slot])

    @pl.when(i == 0)
    def _prologue():
        for d in range(NBUF):
            cx, cn = in_copies(d)
            cx.start()
            cn.start()

    slot = lax.rem(i, NBUF)
    cx, cn = in_copies(i)
    cx.wait()
    cn.wait()

    @pl.when(i >= NBUF)
    def _drain_out():
        out_copy(i - NBUF).wait()

    b = i // cpb
    tt = t_sm[b]
    c1 = c1t[tt]
    c2 = c2t[tt]
    ob[slot] = xb[slot] * c1 + nb[slot] * c2

    out_copy(i).start()

    @pl.when(i + NBUF < num)
    def _next_in():
        cx2, cn2 = in_copies(i + NBUF)
        cx2.start()
        cn2.start()

    @pl.when(i == num - 1)
    def _epilogue():
        for d in range(NBUF):
            out_copy(num - NBUF + d).wait()


def kernel(x_start, t, noise, sqrt_alphas_cumprod, sqrt_one_minus_alphas_cumprod):
    B, C, H, W = x_start.shape
    F = C * H * W
    rows_per_batch = F // 128
    x3 = x_start.reshape(B, rows_per_batch, 128)
    n3 = noise.reshape(B, rows_per_batch, 128)
    cpb = rows_per_batch // ROWS
    num_steps = B * cpb

    any_spec = pl.BlockSpec(memory_space=pl.ANY)
    out = pl.pallas_call(
        _qsample_body,
        grid_spec=pltpu.PrefetchScalarGridSpec(
            num_scalar_prefetch=3,
            grid=(num_steps,),
            in_specs=[any_spec, any_spec],
            out_specs=any_spec,
            scratch_shapes=[
                pltpu.VMEM((NBUF, ROWS, 128), jnp.float32),
                pltpu.VMEM((NBUF, ROWS, 128), jnp.float32),
                pltpu.VMEM((NBUF, ROWS, 128), jnp.float32),
                pltpu.SemaphoreType.DMA((NBUF,)),
                pltpu.SemaphoreType.DMA((NBUF,)),
                pltpu.SemaphoreType.DMA((NBUF,)),
            ],
        ),
        out_shape=jax.ShapeDtypeStruct((B, rows_per_batch, 128), x_start.dtype),
        compiler_params=pltpu.CompilerParams(
            dimension_semantics=("arbitrary",),
        ),
    )(t, sqrt_alphas_cumprod, sqrt_one_minus_alphas_cumprod, x3, n3)
    return out.reshape(B, C, H, W)


# speedup vs baseline: 4.4567x; 4.4567x over previous
import jax
import jax.numpy as jnp
from jax.experimental import pallas as pl
from jax.experimental.pallas import tpu as pltpu

CB = 8


def _qsample_body(t_ref, c1tab_ref, c2tab_ref, x_ref, n_ref, o_ref):
    b = pl.program_id(0)
    tt = t_ref[b]
    c1 = c1tab_ref[tt]
    c2 = c2tab_ref[tt]
    o_ref[...] = x_ref[...] * c1 + n_ref[...] * c2


def kernel(x_start, t, noise, sqrt_alphas_cumprod, sqrt_one_minus_alphas_cumprod):
    B, C, H, W = x_start.shape
    grid = (B, C // CB)

    data_spec = pl.BlockSpec((1, CB, H, W), lambda b, c, *_: (b, c, 0, 0))
    return pl.pallas_call(
        _qsample_body,
        grid_spec=pltpu.PrefetchScalarGridSpec(
            num_scalar_prefetch=3,
            grid=grid,
            in_specs=[data_spec, data_spec],
            out_specs=data_spec,
        ),
        out_shape=jax.ShapeDtypeStruct((B, C, H, W), x_start.dtype),
        compiler_params=pltpu.CompilerParams(
            dimension_semantics=("parallel", "arbitrary"),
        ),
    )(t, sqrt_alphas_cumprod, sqrt_one_minus_alphas_cumprod, x_start, noise)
